# baseline (speedup 1.0000x reference)
import jax
import jax.numpy as jnp
from jax.experimental import pallas as pl
from jax.experimental.pallas import tpu as pltpu

B = 16
T = 4
P = 1025
H = 1280
R = 9


def _body(ids_ref, gate_ref, h_ref, emb_ref, o_ref):
    g = jnp.tanh(gate_ref[0])
    o_ref[...] = h_ref[...] + emb_ref[...] * g


def kernel(hidden_state, aspect_ratio_ids, embedding_weight, gate):
    ids = aspect_ratio_ids.astype(jnp.int32)
    emb = embedding_weight.reshape(R, T, 1, H)

    grid_spec = pltpu.PrefetchScalarGridSpec(
        num_scalar_prefetch=2,
        grid=(B, T),
        in_specs=[
            pl.BlockSpec((1, 1, P, H), lambda b, t, ids, gate: (b, t, 0, 0)),
            pl.BlockSpec((1, 1, 1, H), lambda b, t, ids, gate: (ids[b], t, 0, 0)),
        ],
        out_specs=pl.BlockSpec((1, 1, P, H), lambda b, t, ids, gate: (b, t, 0, 0)),
    )

    return pl.pallas_call(
        _body,
        grid_spec=grid_spec,
        out_shape=jax.ShapeDtypeStruct((B, T, P, H), jnp.float32),
    )(ids, gate, hidden_state, emb)


# speedup vs baseline: 1.0102x; 1.0102x over previous
import jax
import jax.numpy as jnp
from jax.experimental import pallas as pl
from jax.experimental.pallas import tpu as pltpu

B = 16
T = 4
P = 1025
H = 1280
R = 9


def _body(ids_ref, gate_ref, h_ref, emb_ref, o_ref):
    g = jnp.tanh(gate_ref[0])
    o_ref[...] = h_ref[...] + emb_ref[...] * g


def kernel(hidden_state, aspect_ratio_ids, embedding_weight, gate):
    ids = aspect_ratio_ids.astype(jnp.int32)
    emb = embedding_weight.reshape(R, T, 1, H)

    grid_spec = pltpu.PrefetchScalarGridSpec(
        num_scalar_prefetch=2,
        grid=(B, T),
        in_specs=[
            pl.BlockSpec((1, 1, P, H), lambda b, t, ids, gate: (b, t, 0, 0)),
            pl.BlockSpec((1, 1, 1, H), lambda b, t, ids, gate: (ids[b], t, 0, 0)),
        ],
        out_specs=pl.BlockSpec((1, 1, P, H), lambda b, t, ids, gate: (b, t, 0, 0)),
    )

    return pl.pallas_call(
        _body,
        grid_spec=grid_spec,
        out_shape=jax.ShapeDtypeStruct((B, T, P, H), jnp.float32),
        compiler_params=pltpu.CompilerParams(
            dimension_semantics=("parallel", "parallel"),
        ),
    )(ids, gate, hidden_state, emb)
